# baseline (speedup 1.0000x reference)
import jax
import jax.numpy as jnp
from jax import lax
from jax.experimental import pallas as pl
from jax.experimental.pallas import tpu as pltpu

_NUM_TYPES = 87
_CLS = 88
_L = 1024
_C = 8
_R = _C * _L
_K = 6


def _one_hot_block(ids_ref, w, g):
    cls = lax.broadcasted_iota(jnp.int32, (_CLS, _L), 0)
    row = ids_ref[pl.ds(g, 1), :]
    oh_t = (cls == row).astype(jnp.float32)
    return lax.dot_general(oh_t, w, (((0,), (0,)), ((), ())),
                           preferred_element_type=jnp.float32)


def _encode_ring_kernel(s, ids_ref, w_ref, out_hbm, buf, sems):
    j = pl.program_id(1)
    slot = lax.rem(j, _K)
    row0 = (pl.program_id(0) * s + j) * _R

    @pl.when(j >= _K)
    def _():
        pltpu.make_async_copy(buf.at[slot], buf.at[slot], sems.at[slot]).wait()

    w = w_ref[...]
    for g in range(_C):
        buf[slot, pl.ds(g * _L, _L), :] = _one_hot_block(ids_ref, w, g)

    pltpu.make_async_copy(
        buf.at[slot], out_hbm.at[pl.ds(row0, _R), :], sems.at[slot]).start()

    @pl.when(j == s - 1)
    def _():
        for k in range(_K):
            pltpu.make_async_copy(buf.at[k], buf.at[k], sems.at[k]).wait()


def _encode_simple_kernel(ids_ref, w_ref, out_ref):
    w = w_ref[...]
    for g in range(_C):
        out_ref[pl.ds(g * _L, _L), :] = _one_hot_block(ids_ref, w, g)


@jax.jit
def kernel(type_numbers, w_one_hot, electron_config, w_config):
    types = type_numbers.reshape(-1).astype(jnp.int32)
    n = types.shape[0]

    w_comb = (jnp.transpose(w_one_hot)
              + electron_config @ jnp.transpose(w_config)).astype(jnp.float32)
    w_pad = jnp.pad(w_comb, ((0, _CLS - _NUM_TYPES), (0, 0)))

    if n % (2 * _R) == 0 and (n // (2 * _R)) >= _K:
        s = n // (2 * _R)
        ids2d = types.reshape(2 * s * _C, _L)
        return pl.pallas_call(
            lambda *a: _encode_ring_kernel(s, *a),
            out_shape=jax.ShapeDtypeStruct((n, _NUM_TYPES), jnp.float32),
            grid=(2, s),
            in_specs=[
                pl.BlockSpec((_C, _L), lambda c, j, s=s: (c * s + j, 0)),
                pl.BlockSpec((_CLS, _NUM_TYPES), lambda c, j: (0, 0)),
            ],
            out_specs=pl.BlockSpec(memory_space=pl.ANY),
            scratch_shapes=[
                pltpu.VMEM((_K, _R, _NUM_TYPES), jnp.float32),
                pltpu.SemaphoreType.DMA((_K,)),
            ],
            compiler_params=pltpu.CompilerParams(
                dimension_semantics=("parallel", "arbitrary")),
        )(ids2d, w_pad)

    rows = pl.cdiv(n, _L)
    num_steps = pl.cdiv(rows, _C)
    pad = num_steps * _C * _L - n
    if pad:
        types = jnp.pad(types, (0, pad), constant_values=2 ** 30)
    ids2d = types.reshape(num_steps * _C, _L)
    return pl.pallas_call(
        _encode_simple_kernel,
        out_shape=jax.ShapeDtypeStruct((n, _NUM_TYPES), jnp.float32),
        grid=(num_steps,),
        in_specs=[
            pl.BlockSpec((_C, _L), lambda i: (i, 0)),
            pl.BlockSpec((_CLS, _NUM_TYPES), lambda i: (0, 0)),
        ],
        out_specs=pl.BlockSpec((_R, _NUM_TYPES), lambda i: (i, 0)),
        compiler_params=pltpu.CompilerParams(
            dimension_semantics=("parallel",)),
    )(ids2d, w_pad)
